# baseline (speedup 1.0000x reference)
import numpy as np
import jax
import jax.numpy as jnp
from jax.experimental import pallas as pl

_HEAD_INDEX = [1, 2, 3, 4, 5, 6, 7, 8, 9, 10, 11, 12, 13, 17, 18]
_REGIONS = [[1], [17, 18], [4, 5, 6], [2, 3], [7, 8, 9], [10], [12, 13], [11]]
_TEMP = 0.01
_EPS = 1e-8
_NEG = -1e30

_LUT = np.full((19,), -1.0, np.float32)
for _r, _grp in enumerate(_REGIONS):
    for _l in _grp:
        _LUT[_l] = float(_r)


def _corr_kernel(fa_ref, ft_ref, itr_ref, rac_ref, rtr_ref, iac_ref, itm_ref,
                 genh_ref, geni_ref):
    fa = fa_ref[0]
    ft = ft_ref[0]
    itr = itr_ref[0]
    rac = rac_ref[0]
    rtr = rtr_ref[0]
    iac = iac_ref[0]
    itm = itm_ref[0]

    def _norm(x):
        x = x - jnp.mean(x, axis=0, keepdims=True)
        n = jnp.sqrt(jnp.sum(x * x, axis=0, keepdims=True)) + _EPS
        return x / n

    fan = _norm(fa)
    ftn = _norm(ft)
    logits = jax.lax.dot_general(
        fan, ftn, (((0,), (0,)), ((), ())),
        precision=jax.lax.Precision.HIGHEST,
        preferred_element_type=jnp.float32) * (1.0 / _TEMP)

    mh = jnp.logical_and(rac == rtr, rac >= 0.0)
    lh = jnp.where(mh, logits, _NEG)
    mxh = jnp.max(lh, axis=1, keepdims=True)
    ph = jnp.exp(lh - mxh)
    fh = ph / jnp.sum(ph, axis=1, keepdims=True)
    fh = jnp.where(mxh > 0.5 * _NEG, fh, 0.0)
    genh_ref[0] = jax.lax.dot_general(
        itr, fh, (((1,), (1,)), ((), ())),
        precision=jax.lax.Precision.HIGHEST,
        preferred_element_type=jnp.float32)

    li = jnp.where(itm > 0.5, logits, _NEG)
    mxi = jnp.max(li, axis=1, keepdims=True)
    pi = jnp.exp(li - mxi)
    fi = pi / jnp.sum(pi, axis=1, keepdims=True)
    keep = jnp.logical_and(iac > 0.5, mxi > 0.5 * _NEG)
    fi = jnp.where(keep, fi, 0.0)
    geni_ref[0] = jax.lax.dot_general(
        itr, fi, (((1,), (1,)), ((), ())),
        precision=jax.lax.Precision.HIGHEST,
        preferred_element_type=jnp.float32)


def _conv2d(x, w):
    return jax.lax.conv_general_dilated(
        x, w, (1, 1), 'SAME', dimension_numbers=('NCHW', 'OIHW', 'NCHW'))


def _maxpool2(x):
    return jax.lax.reduce_window(x, -jnp.inf, jax.lax.max,
                                 (1, 1, 2, 2), (1, 1, 2, 2), 'VALID')


def _dilate(m, k=3):
    p = k // 2
    return jax.lax.reduce_window(m.astype(jnp.float32), -jnp.inf, jax.lax.max,
                                 (1, 1, k, k), (1, 1, 1, 1),
                                 [(0, 0), (0, 0), (p, p), (p, p)])


def kernel(I_a, I_gray, I_t, M_a, M_t, gt, Wf1, Wf2, Wf3, Wphi, Wth, Wd1, Wd2):
    B, _, H, W = I_a.shape

    x = jnp.concatenate([I_a, I_t], axis=0)
    x = _maxpool2(jax.nn.relu(_conv2d(x, Wf1)))
    x = _maxpool2(jax.nn.relu(_conv2d(x, Wf2)))
    x = _maxpool2(jax.nn.relu(_conv2d(x, Wf3)))
    fA = _conv2d(x[:B], Wphi)
    fT = _conv2d(x[B:], Wth)
    h, w = fA.shape[2], fA.shape[3]
    hw = h * w
    r = H // h

    head = jnp.asarray(_HEAD_INDEX)
    M_Ah = jnp.isin(M_a, head).astype(jnp.float32)
    M_Th = jnp.isin(M_t, head).astype(jnp.float32)
    M_Th_c = jnp.clip(M_Th, 0, 1)
    M_Ti = _dilate(M_Th_c) - M_Th_c
    s = jnp.clip(M_Ah + M_Th, 0, 1)
    M_Ad = _dilate(s)
    M_Ai = M_Ad - M_Ah

    lut = jnp.asarray(_LUT)
    ra = lut[M_a[:, 0, ::r, ::r]].reshape(B, hw)
    rt = lut[M_t[:, 0, ::r, ::r]].reshape(B, hw)
    ia = M_Ai[:, 0, ::r, ::r].reshape(B, hw)
    it = M_Ti[:, 0, ::r, ::r].reshape(B, hw)

    itr = I_t.reshape(B, 3, h, r, w, r).mean(axis=(3, 5)).reshape(B, 3, hw)

    C = fA.shape[1]
    genh, geni = pl.pallas_call(
        _corr_kernel,
        grid=(B,),
        in_specs=[
            pl.BlockSpec((1, C, hw), lambda b: (b, 0, 0)),
            pl.BlockSpec((1, C, hw), lambda b: (b, 0, 0)),
            pl.BlockSpec((1, 3, hw), lambda b: (b, 0, 0)),
            pl.BlockSpec((1, hw, 1), lambda b: (b, 0, 0)),
            pl.BlockSpec((1, 1, hw), lambda b: (b, 0, 0)),
            pl.BlockSpec((1, hw, 1), lambda b: (b, 0, 0)),
            pl.BlockSpec((1, 1, hw), lambda b: (b, 0, 0)),
        ],
        out_specs=[
            pl.BlockSpec((1, 3, hw), lambda b: (b, 0, 0)),
            pl.BlockSpec((1, 3, hw), lambda b: (b, 0, 0)),
        ],
        out_shape=[
            jax.ShapeDtypeStruct((B, 3, hw), jnp.float32),
            jax.ShapeDtypeStruct((B, 3, hw), jnp.float32),
        ],
    )(fA.reshape(B, C, hw), fT.reshape(B, C, hw), itr,
      ra.reshape(B, hw, 1), rt.reshape(B, 1, hw),
      ia.reshape(B, hw, 1), it.reshape(B, 1, hw))

    gen_h = jnp.repeat(jnp.repeat(genh.reshape(B, 3, h, w), r, axis=2), r, axis=3)
    gen_i = jnp.repeat(jnp.repeat(geni.reshape(B, 3, h, w), r, axis=2), r, axis=3)

    I_tb = gt * (1.0 - M_Ad)
    I_ag = I_gray * M_Ah
    inp = jnp.concatenate([gen_h, gen_i, M_Ah, I_tb, M_Ai, I_ag], axis=1)
    oup = _conv2d(jax.nn.relu(_conv2d(inp, Wd1)), Wd2)
    return oup


# speedup vs baseline: 1.0320x; 1.0320x over previous
import numpy as np
import jax
import jax.numpy as jnp
from jax.experimental import pallas as pl

_HEAD_INDEX = [1, 2, 3, 4, 5, 6, 7, 8, 9, 10, 11, 12, 13, 17, 18]
_REGIONS = [[1], [17, 18], [4, 5, 6], [2, 3], [7, 8, 9], [10], [12, 13], [11]]
_TEMP = 0.01
_EPS = 1e-8
_NEG = -1e30

_LUT = np.full((19,), -1.0, np.float32)
for _r, _grp in enumerate(_REGIONS):
    for _l in _grp:
        _LUT[_l] = float(_r)


def _corr_kernel(fa_ref, ft_ref, itr_ref, rar_ref, rtr_ref, iar_ref, itm_ref,
                 genh_ref, geni_ref):
    fa = fa_ref[0]
    ft = ft_ref[0]
    itr = itr_ref[0]
    rtr = rtr_ref[0]
    itm = itm_ref[0]
    rac = jnp.transpose(rar_ref[0])
    iac = jnp.transpose(iar_ref[0])

    def _norm(x):
        x = x - jnp.mean(x, axis=0, keepdims=True)
        n = jnp.sqrt(jnp.sum(x * x, axis=0, keepdims=True)) + _EPS
        return x / n

    fan = _norm(fa)
    ftn = _norm(ft)
    logits = jax.lax.dot_general(
        fan, ftn, (((0,), (0,)), ((), ())),
        precision=jax.lax.Precision.HIGHEST,
        preferred_element_type=jnp.float32) * (1.0 / _TEMP)

    mh = jnp.logical_and(rac == rtr, rac >= 0.0)
    lh = jnp.where(mh, logits, _NEG)
    mxh = jnp.max(lh, axis=1, keepdims=True)
    ph = jnp.exp(lh - mxh)
    fh = ph / jnp.sum(ph, axis=1, keepdims=True)
    fh = jnp.where(mxh > 0.5 * _NEG, fh, 0.0)
    genh_ref[0] = jax.lax.dot_general(
        itr, fh, (((1,), (1,)), ((), ())),
        precision=jax.lax.Precision.HIGHEST,
        preferred_element_type=jnp.float32)

    li = jnp.where(itm > 0.5, logits, _NEG)
    mxi = jnp.max(li, axis=1, keepdims=True)
    pi = jnp.exp(li - mxi)
    fi = pi / jnp.sum(pi, axis=1, keepdims=True)
    keep = jnp.logical_and(iac > 0.5, mxi > 0.5 * _NEG)
    fi = jnp.where(keep, fi, 0.0)
    geni_ref[0] = jax.lax.dot_general(
        itr, fi, (((1,), (1,)), ((), ())),
        precision=jax.lax.Precision.HIGHEST,
        preferred_element_type=jnp.float32)


def _conv2d(x, w):
    return jax.lax.conv_general_dilated(
        x, w, (1, 1), 'SAME', dimension_numbers=('NCHW', 'OIHW', 'NCHW'))


def _maxpool2(x):
    return jax.lax.reduce_window(x, -jnp.inf, jax.lax.max,
                                 (1, 1, 2, 2), (1, 1, 2, 2), 'VALID')


def _dilate(m, k=3):
    p = k // 2
    return jax.lax.reduce_window(m.astype(jnp.float32), -jnp.inf, jax.lax.max,
                                 (1, 1, k, k), (1, 1, 1, 1),
                                 [(0, 0), (0, 0), (p, p), (p, p)])


def kernel(I_a, I_gray, I_t, M_a, M_t, gt, Wf1, Wf2, Wf3, Wphi, Wth, Wd1, Wd2):
    B, _, H, W = I_a.shape

    x = jnp.concatenate([I_a, I_t], axis=0)
    x = _maxpool2(jax.nn.relu(_conv2d(x, Wf1)))
    x = _maxpool2(jax.nn.relu(_conv2d(x, Wf2)))
    x = _maxpool2(jax.nn.relu(_conv2d(x, Wf3)))
    fA = _conv2d(x[:B], Wphi)
    fT = _conv2d(x[B:], Wth)
    h, w = fA.shape[2], fA.shape[3]
    hw = h * w
    r = H // h

    head = jnp.asarray(_HEAD_INDEX)
    M_Ah = jnp.isin(M_a, head).astype(jnp.float32)
    M_Th = jnp.isin(M_t, head).astype(jnp.float32)
    M_Th_c = jnp.clip(M_Th, 0, 1)
    M_Ti = _dilate(M_Th_c) - M_Th_c
    s = jnp.clip(M_Ah + M_Th, 0, 1)
    M_Ad = _dilate(s)
    M_Ai = M_Ad - M_Ah

    def _region_id(lbl):
        rid = jnp.full(lbl.shape, -1.0, jnp.float32)
        for ridx, grp in enumerate(_REGIONS):
            hit = lbl == grp[0]
            for g in grp[1:]:
                hit = jnp.logical_or(hit, lbl == g)
            rid = jnp.where(hit, float(ridx), rid)
        return rid

    ra = _region_id(M_a[:, 0, ::r, ::r]).reshape(B, hw)
    rt = _region_id(M_t[:, 0, ::r, ::r]).reshape(B, hw)
    ia = M_Ai[:, 0, ::r, ::r].reshape(B, hw)
    it = M_Ti[:, 0, ::r, ::r].reshape(B, hw)

    itr = I_t.reshape(B, 3, h, r, w, r).mean(axis=(3, 5)).reshape(B, 3, hw)

    C = fA.shape[1]
    genh, geni = pl.pallas_call(
        _corr_kernel,
        grid=(B,),
        in_specs=[
            pl.BlockSpec((1, C, hw), lambda b: (b, 0, 0)),
            pl.BlockSpec((1, C, hw), lambda b: (b, 0, 0)),
            pl.BlockSpec((1, 3, hw), lambda b: (b, 0, 0)),
            pl.BlockSpec((1, 1, hw), lambda b: (b, 0, 0)),
            pl.BlockSpec((1, 1, hw), lambda b: (b, 0, 0)),
            pl.BlockSpec((1, 1, hw), lambda b: (b, 0, 0)),
            pl.BlockSpec((1, 1, hw), lambda b: (b, 0, 0)),
        ],
        out_specs=[
            pl.BlockSpec((1, 3, hw), lambda b: (b, 0, 0)),
            pl.BlockSpec((1, 3, hw), lambda b: (b, 0, 0)),
        ],
        out_shape=[
            jax.ShapeDtypeStruct((B, 3, hw), jnp.float32),
            jax.ShapeDtypeStruct((B, 3, hw), jnp.float32),
        ],
    )(fA.reshape(B, C, hw), fT.reshape(B, C, hw), itr,
      ra.reshape(B, 1, hw), rt.reshape(B, 1, hw),
      ia.reshape(B, 1, hw), it.reshape(B, 1, hw))

    gen_h = jnp.repeat(jnp.repeat(genh.reshape(B, 3, h, w), r, axis=2), r, axis=3)
    gen_i = jnp.repeat(jnp.repeat(geni.reshape(B, 3, h, w), r, axis=2), r, axis=3)

    I_tb = gt * (1.0 - M_Ad)
    I_ag = I_gray * M_Ah
    inp = jnp.concatenate([gen_h, gen_i, M_Ah, I_tb, M_Ai, I_ag], axis=1)
    oup = _conv2d(jax.nn.relu(_conv2d(inp, Wd1)), Wd2)
    return oup
